# baseline (speedup 1.0000x reference)
import functools

import jax
import jax.numpy as jnp
from jax.experimental import pallas as pl
from jax.experimental.pallas import tpu as pltpu


def _seq_mean_kernel(x_ref, o_ref, *, inv_seq_len):
    x = x_ref[...].astype(jnp.float32)
    s = jnp.sum(x, axis=1) * inv_seq_len
    o_ref[...] = s[None].astype(o_ref.dtype)


def kernel(x):
    B, S, H = x.shape
    itemsize = jnp.dtype(x.dtype).itemsize

    max_block_bytes = 4 * 1024 * 1024
    tb = max(1, max_block_bytes // (S * H * itemsize))
    while B % tb:
        tb -= 1

    n_blk = B // tb
    kernel_fn = functools.partial(_seq_mean_kernel, inv_seq_len=1.0 / S)
    out = pl.pallas_call(
        kernel_fn,
        out_shape=jax.ShapeDtypeStruct((n_blk, tb, H), x.dtype),
        grid=(n_blk,),
        in_specs=[pl.BlockSpec((tb, S, H), lambda b: (b, 0, 0))],
        out_specs=pl.BlockSpec((1, tb, H), lambda b: (b, 0, 0)),
        compiler_params=pltpu.CompilerParams(
            dimension_semantics=("parallel",),
            vmem_limit_bytes=48 * 1024 * 1024,
        ),
    )(x)
    return out.reshape(B, H)


# speedup vs baseline: 1.1901x; 1.1901x over previous
import functools

import jax
import jax.numpy as jnp
from jax.experimental import pallas as pl
from jax.experimental.pallas import tpu as pltpu


def _seq_mean_kernel(x_ref, o_ref, *, inv_seq_len):
    x = x_ref[...].astype(jnp.float32)
    s = jnp.sum(x, axis=1) * inv_seq_len
    o_ref[...] = s[None].astype(o_ref.dtype)


def kernel(x):
    B, S, H = x.shape
    itemsize = jnp.dtype(x.dtype).itemsize

    max_block_bytes = 26 * 1024 * 1024
    tb = max(1, max_block_bytes // (S * H * itemsize))
    while B % tb:
        tb -= 1

    n_blk = B // tb
    kernel_fn = functools.partial(_seq_mean_kernel, inv_seq_len=1.0 / S)
    out = pl.pallas_call(
        kernel_fn,
        out_shape=jax.ShapeDtypeStruct((n_blk, tb, H), x.dtype),
        grid=(n_blk,),
        in_specs=[pl.BlockSpec((tb, S, H), lambda b: (b, 0, 0))],
        out_specs=pl.BlockSpec((1, tb, H), lambda b: (b, 0, 0)),
        compiler_params=pltpu.CompilerParams(
            dimension_semantics=("parallel",),
            vmem_limit_bytes=60 * 1024 * 1024,
        ),
    )(x)
    return out.reshape(B, H)


# speedup vs baseline: 1.1993x; 1.0077x over previous
import functools

import jax
import jax.numpy as jnp
from jax.experimental import pallas as pl
from jax.experimental.pallas import tpu as pltpu


def _mean_pipeline_kernel(x_hbm, o_ref, x_buf, in_sem, *, tb, nbuf, n_chunks,
                          inv_seq_len):
    def start(c):
        pltpu.make_async_copy(
            x_hbm.at[pl.ds(c * tb, tb)], x_buf.at[c % nbuf], in_sem.at[c % nbuf]
        ).start()

    def wait(c):
        pltpu.make_async_copy(
            x_hbm.at[pl.ds(0, tb)], x_buf.at[c % nbuf], in_sem.at[c % nbuf]
        ).wait()

    for c in range(min(nbuf - 1, n_chunks)):
        start(c)
    for c in range(n_chunks):
        if c + nbuf - 1 < n_chunks:
            start(c + nbuf - 1)
        wait(c)
        x = x_buf[c % nbuf].astype(jnp.float32)
        o_ref[pl.ds(c * tb, tb), :] = (
            jnp.sum(x, axis=1) * inv_seq_len).astype(o_ref.dtype)


def kernel(x):
    B, S, H = x.shape
    itemsize = jnp.dtype(x.dtype).itemsize

    max_chunk_bytes = 13 * 1024 * 1024
    tb = max(1, max_chunk_bytes // (S * H * itemsize))
    while B % tb:
        tb -= 1
    nbuf = 3
    n_chunks = B // tb

    kernel_fn = functools.partial(
        _mean_pipeline_kernel, tb=tb, nbuf=nbuf, n_chunks=n_chunks,
        inv_seq_len=1.0 / S)
    return pl.pallas_call(
        kernel_fn,
        out_shape=jax.ShapeDtypeStruct((B, H), x.dtype),
        in_specs=[pl.BlockSpec(memory_space=pl.ANY)],
        out_specs=pl.BlockSpec((B, H), lambda: (0, 0)),
        scratch_shapes=[
            pltpu.VMEM((nbuf, tb, S, H), x.dtype),
            pltpu.SemaphoreType.DMA((nbuf,)),
        ],
        compiler_params=pltpu.CompilerParams(
            vmem_limit_bytes=56 * 1024 * 1024,
        ),
    )(x)


# speedup vs baseline: 1.1995x; 1.0002x over previous
import functools

import jax
import jax.numpy as jnp
from jax.experimental import pallas as pl
from jax.experimental.pallas import tpu as pltpu


def _seq_mean_kernel(x_ref, o_ref, *, inv_seq_len):
    x = x_ref[...].astype(jnp.float32)
    o_ref[...] = (jnp.sum(x, axis=1) * inv_seq_len).astype(o_ref.dtype)


def kernel(x):
    B, S, H = x.shape
    itemsize = jnp.dtype(x.dtype).itemsize

    max_block_bytes = 13 * 1024 * 1024
    tb = max(1, max_block_bytes // (S * H * itemsize))
    while B % tb:
        tb -= 1

    grid = (B // tb,)
    kernel_fn = functools.partial(_seq_mean_kernel, inv_seq_len=1.0 / S)
    return pl.pallas_call(
        kernel_fn,
        out_shape=jax.ShapeDtypeStruct((B, H), x.dtype),
        grid=grid,
        in_specs=[pl.BlockSpec((tb, S, H), lambda b: (b, 0, 0))],
        out_specs=pl.BlockSpec((tb, H), lambda b: (b, 0)),
        compiler_params=pltpu.CompilerParams(
            dimension_semantics=("parallel",),
            vmem_limit_bytes=48 * 1024 * 1024,
        ),
    )(x)
